# baseline (speedup 1.0000x reference)
import jax
import jax.numpy as jnp
from jax import lax
from jax.experimental import pallas as pl
from jax.experimental.pallas import tpu as pltpu
from jax.experimental.pallas import tpu_sc as plsc

_NUM_RANKS = 1024
_MAX_TOK = 77
_D = 768
_CTX = 16
_TPR = 4
_LANES = 16
_NCOL = _D // _LANES

_INFO = plsc.get_sparse_core_info()
_NC = _INFO.num_cores
_NS = _INFO.num_subcores
_NW = _NC * _NS
_RPP = _NUM_RANKS // _NW

_T0 = 1 + _CTX + _TPR
_NT = _MAX_TOK - _T0


def _sc_body(ctx_hbm, rank_hbm, sent_hbm, out_hbm, sbuf, cbuf, rkbuf,
             si_sem, so_sem, co_sem, cx_sem, rk_sem, q_sem):
    core = lax.axis_index("c")
    sid = lax.axis_index("s")
    wid = sid * _NC + core
    base = wid * _RPP

    def sin(t, sl):
        return pltpu.make_async_copy(
            sent_hbm.at[t].at[pl.ds(base, _RPP)], sbuf.at[sl],
            si_sem.at[sl])

    def sout(t, sl):
        return pltpu.make_async_copy(
            sbuf.at[sl], out_hbm.at[t].at[pl.ds(base, _RPP)],
            so_sem.at[sl])

    def cout(t, sl):
        return pltpu.make_async_copy(
            cbuf.at[sl], out_hbm.at[1 + t].at[pl.ds(base, _RPP)],
            co_sem.at[sl])

    def ctxin(sl):
        return pltpu.make_async_copy(
            ctx_hbm, cbuf.at[sl].at[pl.ds(0, _CTX)], cx_sem.at[sl])

    def rkin(g):
        return pltpu.make_async_copy(
            rank_hbm.at[pl.ds(base + 8 * g, 8)], rkbuf, rk_sem)

    def qout(g, j, sl):
        return pltpu.make_async_copy(
            cbuf.at[sl].at[pl.ds(0, 8)],
            out_hbm.at[1 + _CTX + j].at[pl.ds(base + 8 * g, 8)],
            q_sem.at[sl])

    ctxin(0).start()
    ctxin(1).start()
    sin(_T0, 0).start()

    def step(i, carry):
        t = _T0 + i
        sl = lax.rem(i, 2)

        sin(t, sl).wait()

        @pl.when(i + 1 < _NT)
        def _lookahead():
            @pl.when(i >= 1)
            def _reclaim():
                sout(t - 1, 1 - sl).wait()

            sin(t + 1, 1 - sl).start()

        sout(t, sl).start()

        @pl.when(i < _CTX)
        def _ctx_plane():
            @pl.when(i >= 2)
            def _free():
                cout(i - 2, sl).wait()
                ctxin(sl).start()

            ctxin(sl).wait()
            c = cbuf.at[sl]
            for k in range(_NCOL):
                c[_RPP - 1, pl.ds(k * _LANES, _LANES)] = (
                    c[i, pl.ds(k * _LANES, _LANES)])
            for rr in range(_RPP - 1):
                for k in range(_NCOL):
                    c[rr, pl.ds(k * _LANES, _LANES)] = (
                        c[_RPP - 1, pl.ds(k * _LANES, _LANES)])
            cout(i, sl).start()

        return carry

    lax.fori_loop(0, _NT, step, 0)

    cout(_CTX - 2, lax.rem(_CTX - 2, 2)).wait()
    cout(_CTX - 1, lax.rem(_CTX - 1, 2)).wait()
    sout(_MAX_TOK - 2, lax.rem(_NT - 2, 2)).wait()
    sout(_MAX_TOK - 1, lax.rem(_NT - 1, 2)).wait()

    sin(0, 0).start()
    rkin(0).start()
    sin(0, 0).wait()
    sout(0, 0).start()
    qn = 0
    for g in range(_TPR):
        rkin(g).wait()
        for j in range(_TPR):
            sl = qn % 2
            if qn >= 2:
                p2 = qn - 2
                qout(p2 // _TPR, p2 % _TPR, p2 % 2).wait()
            for rr in range(8):
                for k in range(_NCOL):
                    cbuf[sl, rr, pl.ds(k * _LANES, _LANES)] = (
                        rkbuf[rr, j, pl.ds(k * _LANES, _LANES)])
            qout(g, j, sl).start()
            qn += 1
        if g + 1 < _TPR:
            rkin(g + 1).start()
    qout(3, 2, 0).wait()
    qout(3, 3, 1).wait()
    sout(0, 0).wait()


def kernel(context_embeds, rank_embeds, sentence_embeds):
    sent_t = jnp.transpose(sentence_embeds, (1, 0, 2))
    mesh = plsc.VectorSubcoreMesh(core_axis_name="c", subcore_axis_name="s")
    k = pl.kernel(
        _sc_body,
        out_type=jax.ShapeDtypeStruct((_MAX_TOK, _NUM_RANKS, _D),
                                      jnp.float32),
        mesh=mesh,
        scratch_types=[
            pltpu.VMEM((2, _RPP, _D), jnp.float32),
            pltpu.VMEM((2, _RPP, _D), jnp.float32),
            pltpu.VMEM((8, _TPR, _D), jnp.float32),
            pltpu.SemaphoreType.DMA((2,)),
            pltpu.SemaphoreType.DMA((2,)),
            pltpu.SemaphoreType.DMA((2,)),
            pltpu.SemaphoreType.DMA((2,)),
            pltpu.SemaphoreType.DMA,
            pltpu.SemaphoreType.DMA((2,)),
        ],
    )
    out_t = k(context_embeds, rank_embeds, sent_t)
    return jnp.transpose(out_t, (1, 0, 2))


# speedup vs baseline: 1.3795x; 1.3795x over previous
import jax
import jax.numpy as jnp
from jax import lax
from jax.experimental import pallas as pl
from jax.experimental.pallas import tpu as pltpu
from jax.experimental.pallas import tpu_sc as plsc

_NUM_RANKS = 1024
_MAX_TOK = 77
_D = 768
_CTX = 16
_TPR = 4
_LANES = 16
_NCOL = _D // _LANES

_INFO = plsc.get_sparse_core_info()
_NC = _INFO.num_cores
_NS = _INFO.num_subcores
_NW = _NC * _NS
_RPP = _NUM_RANKS // _NW

_T0 = 1 + _CTX + _TPR
_NT = _MAX_TOK - _T0


def _sc_body(ctx_hbm, rank_hbm, sent_hbm, out_hbm, ctxb, sbuf, cbuf,
             si_sem, so_sem, co_sem):
    core = lax.axis_index("c")
    sid = lax.axis_index("s")
    wid = sid * _NC + core
    base = wid * _RPP

    def sin(t, sl):
        return pltpu.make_async_copy(
            sent_hbm.at[t].at[pl.ds(base, _RPP)], sbuf.at[sl],
            si_sem.at[sl])

    def sout(t, sl):
        return pltpu.make_async_copy(
            sbuf.at[sl], out_hbm.at[t].at[pl.ds(base, _RPP)],
            so_sem.at[sl])

    def rin(j, sl):
        return pltpu.make_async_copy(
            rank_hbm.at[j].at[pl.ds(base, _RPP)], sbuf.at[sl],
            si_sem.at[sl])

    def rout(j, sl):
        return pltpu.make_async_copy(
            sbuf.at[sl], out_hbm.at[1 + _CTX + j].at[pl.ds(base, _RPP)],
            so_sem.at[sl])

    def cout(t, sl):
        return pltpu.make_async_copy(
            cbuf.at[sl], out_hbm.at[1 + t].at[pl.ds(base, _RPP)],
            co_sem.at[sl])

    pltpu.sync_copy(ctx_hbm, ctxb)
    sin(_T0, 0).start()

    def step(i, carry):
        t = _T0 + i
        sl = lax.rem(i, 2)

        sin(t, sl).wait()

        @pl.when(i + 1 < _NT)
        def _lookahead():
            @pl.when(i >= 1)
            def _reclaim():
                sout(t - 1, 1 - sl).wait()

            sin(t + 1, 1 - sl).start()

        sout(t, sl).start()

        @pl.when(i < _CTX)
        def _ctx_plane():
            @pl.when(i >= 2)
            def _free():
                cout(i - 2, sl).wait()

            c = cbuf.at[sl]
            for k in range(_NCOL):
                c[0, pl.ds(k * _LANES, _LANES)] = (
                    ctxb[i, pl.ds(k * _LANES, _LANES)])
            for rr in range(1, _RPP):
                for k in range(_NCOL):
                    c[rr, pl.ds(k * _LANES, _LANES)] = (
                        c[0, pl.ds(k * _LANES, _LANES)])
            cout(i, sl).start()

        return carry

    lax.fori_loop(0, _NT, step, 0)

    cout(_CTX - 2, lax.rem(_CTX - 2, 2)).wait()
    cout(_CTX - 1, lax.rem(_CTX - 1, 2)).wait()
    sout(_MAX_TOK - 2, lax.rem(_NT - 2, 2)).wait()
    sout(_MAX_TOK - 1, lax.rem(_NT - 1, 2)).wait()

    def jin(j, sl):
        return sin(0, sl) if j == 0 else rin(j - 1, sl)

    def jout(j, sl):
        return sout(0, sl) if j == 0 else rout(j - 1, sl)

    jin(0, 0).start()
    jin(1, 1).start()
    for j in range(5):
        sl = j % 2
        jin(j, sl).wait()
        jout(j, sl).start()
        if j + 2 < 5:
            jout(j, sl).wait()
            jin(j + 2, sl).start()
    jout(3, 1).wait()
    jout(4, 0).wait()


def kernel(context_embeds, rank_embeds, sentence_embeds):
    sent_t = jnp.transpose(sentence_embeds, (1, 0, 2))
    rank_t = jnp.transpose(rank_embeds, (1, 0, 2))
    mesh = plsc.VectorSubcoreMesh(core_axis_name="c", subcore_axis_name="s")
    k = pl.kernel(
        _sc_body,
        out_type=jax.ShapeDtypeStruct((_MAX_TOK, _NUM_RANKS, _D),
                                      jnp.float32),
        mesh=mesh,
        scratch_types=[
            pltpu.VMEM((_CTX, _D), jnp.float32),
            pltpu.VMEM((2, _RPP, _D), jnp.float32),
            pltpu.VMEM((2, _RPP, _D), jnp.float32),
            pltpu.SemaphoreType.DMA((2,)),
            pltpu.SemaphoreType.DMA((2,)),
            pltpu.SemaphoreType.DMA((2,)),
        ],
    )
    out_t = k(context_embeds, rank_t, sent_t)
    return jnp.transpose(out_t, (1, 0, 2))
